# baseline (speedup 1.0000x reference)
import jax
import jax.numpy as jnp
from jax.experimental import pallas as pl
from jax.experimental.pallas import tpu as pltpu

_EPS = 1e-8
_HALF = 4096
_N = 8192
_D = 256
_BR = 256
_NB = _N // _BR
_NBH = _HALF // _BR
_CHUNK = 1024
_NC = _N // _CHUNK
_SUB = _CHUNK // _BR
_NORM_BLK = 512
_C = 2.8853900817779268
_SQRT_C = 1.6986436287041668


def _blk(zn_ref, cb):
    return zn_ref[cb // _NBH, pl.ds((cb % _NBH) * _BR, _BR), :]


def _ntxent_kernel(zi_ref, zj_ref, out_ref, zn_ref, acc_ref, rs_ref):
    i = pl.program_id(0)

    @pl.when(i == 0)
    def _normalize():
        for h, ref in enumerate((zi_ref, zj_ref)):
            for k in range(_HALF // _NORM_BLK):
                z = ref[k * _NORM_BLK:(k + 1) * _NORM_BLK, :]
                nrm = jnp.sqrt(jnp.sum(z * z, axis=1, keepdims=True))
                scl = _SQRT_C / jnp.maximum(nrm, _EPS)
                zn_ref[h, k * _NORM_BLK:(k + 1) * _NORM_BLK, :] = (
                    (z * scl).astype(jnp.bfloat16))
        acc_ref[...] = jnp.zeros_like(acc_ref)

    rows = _blk(zn_ref, i)
    pair = zn_ref[1 - i // _NBH, pl.ds((i % _NBH) * _BR, _BR), :]
    rows_f = rows.astype(jnp.float32)
    pos_c = jnp.sum(rows_f * pair.astype(jnp.float32),
                    axis=1, keepdims=True)
    self_c = jnp.sum(rows_f * rows_f, axis=1, keepdims=True)
    rs_ref[...] = jnp.zeros_like(rs_ref)

    c0 = i // _SUB
    r_in = i % _SUB

    for sub in range(_SUB):
        @pl.when(sub >= r_in)
        def _diag_sub(sub=sub):
            cb = c0 * _SUB + sub
            s = jax.lax.dot_general(
                rows, _blk(zn_ref, cb), (((1,), (1,)), ((), ())),
                preferred_element_type=jnp.float32)
            e = jnp.exp2(s)
            rs_ref[...] += jnp.sum(e, axis=1, keepdims=True)

            @pl.when(sub > r_in)
            def _credit():
                acc_ref[cb] += jnp.sum(e, axis=0, keepdims=True)

    for c in range(1, _NC):
        @pl.when(c > c0)
        def _right(c=c):
            chunk = zn_ref[c // (_NC // 2),
                           (c % (_NC // 2)) * _CHUNK:
                           (c % (_NC // 2) + 1) * _CHUNK, :]
            s = jax.lax.dot_general(
                rows, chunk, (((1,), (1,)), ((), ())),
                preferred_element_type=jnp.float32)
            e = jnp.exp2(s)
            rs_ref[...] += jnp.sum(e, axis=1, keepdims=True)
            cs = jnp.sum(e, axis=0, keepdims=True)
            for sub in range(_SUB):
                acc_ref[c * _SUB + sub] += cs[:, sub * _BR:(sub + 1) * _BR]

    credit = jnp.transpose(acc_ref[i], (1, 0))
    total = rs_ref[...] + credit
    out_ref[...] = jnp.log(total - jnp.exp2(self_c)) - (2.0 / _C) * pos_c


def kernel(z_i, z_j):
    per_row = pl.pallas_call(
        _ntxent_kernel,
        out_shape=jax.ShapeDtypeStruct((_N, 1), jnp.float32),
        grid=(_NB,),
        in_specs=[
            pl.BlockSpec((_HALF, _D), lambda i: (0, 0)),
            pl.BlockSpec((_HALF, _D), lambda i: (0, 0)),
        ],
        out_specs=pl.BlockSpec((_BR, 1), lambda i: (i, 0)),
        scratch_shapes=[
            pltpu.VMEM((2, _HALF, _D), jnp.bfloat16),
            pltpu.VMEM((_NB, 1, _BR), jnp.float32),
            pltpu.VMEM((_BR, 1), jnp.float32),
        ],
        compiler_params=pltpu.CompilerParams(
            dimension_semantics=("arbitrary",),
            vmem_limit_bytes=50 * 1024 * 1024),
        name="ntxent_loss",
    )(z_i, z_j)
    return jnp.mean(per_row)


# speedup vs baseline: 2.3364x; 2.3364x over previous
import jax
import jax.numpy as jnp
from jax.experimental import pallas as pl
from jax.experimental.pallas import tpu as pltpu

_EPS = 1e-8
_HALF = 4096
_N = 8192
_D = 256
_BR = 256
_NB = _N // _BR
_NBH = _HALF // _BR
_NORM_BLK = 512
_C = 2.8853900817779268
_SQRT_C = 1.6986436287041668


def _blk(zn_ref, cb):
    return zn_ref[cb // _NBH, pl.ds((cb % _NBH) * _BR, _BR), :]


def _row_terms(zn_ref, rb):
    rows_f = _blk(zn_ref, rb).astype(jnp.float32)
    pair_f = _blk(zn_ref, (rb + _NBH) % _NB).astype(jnp.float32)
    pos_c = jnp.sum(rows_f * pair_f, axis=1, keepdims=True)
    self_c = jnp.sum(rows_f * rows_f, axis=1, keepdims=True)
    return pos_c, self_c


def _finish(total, pos_c, self_c):
    return jnp.log(total - jnp.exp2(self_c)) - (2.0 / _C) * pos_c


def _ntxent_kernel(zi_ref, zj_ref, out_ref, zn_ref, acc_ref):
    i = pl.program_id(0)

    @pl.when(i == 0)
    def _normalize():
        for h, ref in enumerate((zi_ref, zj_ref)):
            for k in range(_HALF // _NORM_BLK):
                z = ref[k * _NORM_BLK:(k + 1) * _NORM_BLK, :]
                nrm = jnp.sqrt(jnp.sum(z * z, axis=1, keepdims=True))
                scl = _SQRT_C / jnp.maximum(nrm, _EPS)
                zn_ref[h, k * _NORM_BLK:(k + 1) * _NORM_BLK, :] = (
                    (z * scl).astype(jnp.bfloat16))
        acc_ref[...] = jnp.zeros_like(acc_ref)

    ra = i
    rbb = _NB - 1 - i
    rs_a = jnp.zeros((_BR, 1), jnp.float32)
    rs_b = jnp.zeros((_BR, 1), jnp.float32)
    for t in range(_NB + 1):
        is_a = ra + t <= _NB - 1
        rb = jnp.where(is_a, ra, rbb)
        cb = jnp.where(is_a, ra + t, t - 1)
        rows = _blk(zn_ref, rb)
        s = jax.lax.dot_general(
            rows, _blk(zn_ref, cb), (((1,), (1,)), ((), ())),
            preferred_element_type=jnp.float32)
        e = jnp.exp2(s)
        rsum = jnp.sum(e, axis=1, keepdims=True)
        rs_a = rs_a + jnp.where(is_a, rsum, 0.0)
        rs_b = rs_b + jnp.where(is_a, 0.0, rsum)
        credit = jnp.where(cb > rb, 1.0, 0.0)
        acc_ref[cb] += credit * jnp.sum(e, axis=0, keepdims=True)

    pos_a, self_a = _row_terms(zn_ref, ra)
    total_a = rs_a + jnp.transpose(acc_ref[ra], (1, 0))
    out_ref[pl.ds(ra * _BR, _BR), :] = _finish(total_a, pos_a, self_a)
    acc_ref[rbb] += jnp.transpose(rs_b, (1, 0))

    @pl.when(i == _NB // 2 - 1)
    def _emit_tail():
        for m in range(_NB // 2, _NB):
            pos_m, self_m = _row_terms(zn_ref, m)
            total_m = jnp.transpose(acc_ref[m], (1, 0))
            out_ref[m * _BR:(m + 1) * _BR, :] = _finish(
                total_m, pos_m, self_m)


def kernel(z_i, z_j):
    per_row = pl.pallas_call(
        _ntxent_kernel,
        out_shape=jax.ShapeDtypeStruct((_N, 1), jnp.float32),
        grid=(_NB // 2,),
        in_specs=[
            pl.BlockSpec((_HALF, _D), lambda i: (0, 0)),
            pl.BlockSpec((_HALF, _D), lambda i: (0, 0)),
        ],
        out_specs=pl.BlockSpec((_N, 1), lambda i: (0, 0)),
        scratch_shapes=[
            pltpu.VMEM((2, _HALF, _D), jnp.bfloat16),
            pltpu.VMEM((_NB, 1, _BR), jnp.float32),
        ],
        compiler_params=pltpu.CompilerParams(
            dimension_semantics=("arbitrary",),
            vmem_limit_bytes=50 * 1024 * 1024),
        name="ntxent_loss",
    )(z_i, z_j)
    return jnp.mean(per_row)


# speedup vs baseline: 2.8227x; 1.2081x over previous
import jax
import jax.numpy as jnp
from jax.experimental import pallas as pl
from jax.experimental.pallas import tpu as pltpu

_EPS = 1e-8
_HALF = 4096
_N = 8192
_D = 256
_BR = 512
_NB = _N // _BR
_NBH = _HALF // _BR
_LT = _BR // 128
_NORM_BLK = 512
_C = 2.8853900817779268
_SQRT_C = 1.6986436287041668


def _blk(zn_ref, b):
    return zn_ref[b // _NBH, pl.ds((b % _NBH) * _BR, _BR), :]


def _emit(zn_ref, rsacc_ref, colacc_ref, out_ref, m):
    rows_f = _blk(zn_ref, m).astype(jnp.float32)
    pair_f = _blk(zn_ref, (m + _NBH) % _NB).astype(jnp.float32)
    pos_c = jnp.sum(rows_f * pair_f, axis=1, keepdims=True)
    self_c = jnp.sum(rows_f * rows_f, axis=1, keepdims=True)
    total = (jnp.sum(rsacc_ref[m], axis=1, keepdims=True)
             + jnp.transpose(colacc_ref[m], (1, 0)))
    out_ref[pl.ds(m * _BR, _BR), :] = (
        jnp.log(total - jnp.exp2(self_c)) - (2.0 / _C) * pos_c)


def _ntxent_kernel(zi_ref, zj_ref, out_ref, zn_ref, rsacc_ref, colacc_ref):
    i = pl.program_id(0)

    @pl.when(i == 0)
    def _init():
        for h, ref in enumerate((zi_ref, zj_ref)):
            for k in range(_HALF // _NORM_BLK):
                z = ref[k * _NORM_BLK:(k + 1) * _NORM_BLK, :]
                nrm = jnp.sqrt(jnp.sum(z * z, axis=1, keepdims=True))
                scl = _SQRT_C / jnp.maximum(nrm, _EPS)
                zn_ref[h, k * _NORM_BLK:(k + 1) * _NORM_BLK, :] = (
                    (z * scl).astype(jnp.bfloat16))
        rsacc_ref[...] = jnp.zeros_like(rsacc_ref)
        colacc_ref[...] = jnp.zeros_like(colacc_ref)

    ra = i
    rbb = _NB - 1 - i
    for t in range(_NB + 1):
        is_a = ra + t <= _NB - 1
        rb = jnp.where(is_a, ra, rbb)
        cb = jnp.where(is_a, ra + t, t - 1)
        s = jax.lax.dot_general(
            _blk(zn_ref, rb), _blk(zn_ref, cb), (((1,), (1,)), ((), ())),
            preferred_element_type=jnp.float32)
        e = jnp.exp2(s)
        part = e[:, 0:128]
        for lt in range(1, _LT):
            part = part + e[:, lt * 128:(lt + 1) * 128]
        rsacc_ref[rb] += part
        credit = jnp.where(cb > rb, 1.0, 0.0)
        colacc_ref[cb] += credit * jnp.sum(e, axis=0, keepdims=True)

    _emit(zn_ref, rsacc_ref, colacc_ref, out_ref, ra)

    @pl.when(i == _NB // 2 - 1)
    def _emit_tail():
        for m in range(_NB // 2, _NB):
            _emit(zn_ref, rsacc_ref, colacc_ref, out_ref, m)


def kernel(z_i, z_j):
    per_row = pl.pallas_call(
        _ntxent_kernel,
        out_shape=jax.ShapeDtypeStruct((_N, 1), jnp.float32),
        grid=(_NB // 2,),
        in_specs=[
            pl.BlockSpec((_HALF, _D), lambda i: (0, 0)),
            pl.BlockSpec((_HALF, _D), lambda i: (0, 0)),
        ],
        out_specs=pl.BlockSpec((_N, 1), lambda i: (0, 0)),
        scratch_shapes=[
            pltpu.VMEM((2, _HALF, _D), jnp.bfloat16),
            pltpu.VMEM((_NB, _BR, 128), jnp.float32),
            pltpu.VMEM((_NB, 1, _BR), jnp.float32),
        ],
        compiler_params=pltpu.CompilerParams(
            dimension_semantics=("arbitrary",),
            vmem_limit_bytes=50 * 1024 * 1024),
        name="ntxent_loss",
    )(z_i, z_j)
    return jnp.mean(per_row)
